# baseline (speedup 1.0000x reference)
import functools

import jax
import jax.numpy as jnp
from jax import lax
from jax.experimental import pallas as pl
from jax.experimental.pallas import tpu as pltpu
from jax.experimental.pallas import tpu_sc as plsc

ENT_DIM = 64
NB = 128


def _sc_gather(table2, idx):
    n = idx.shape[0]
    NW = 32
    CHUNK = n // NW
    mesh = plsc.VectorSubcoreMesh(core_axis_name="c", subcore_axis_name="s")

    @functools.partial(
        pl.kernel,
        mesh=mesh,
        out_type=jax.ShapeDtypeStruct((n, 2 * ENT_DIM), jnp.float32),
        scratch_types=[
            pltpu.VMEM((CHUNK,), jnp.int32),
            pltpu.VMEM((CHUNK, 2 * ENT_DIM), jnp.float32),
            pltpu.SemaphoreType.DMA,
        ],
    )
    def k(tab_hbm, idx_hbm, out_hbm, idx_v, rows_v, sem):
        wid = lax.axis_index("s") * 2 + lax.axis_index("c")
        base = wid * CHUNK
        pltpu.sync_copy(idx_hbm.at[pl.ds(base, CHUNK)], idx_v)
        pltpu.async_copy(tab_hbm.at[idx_v], rows_v, sem).wait()
        pltpu.sync_copy(rows_v, out_hbm.at[pl.ds(base, CHUNK)])

    return k(table2, idx)


def _sel_half(x2, par_i32):
    par = (par_i32 & 1).astype(jnp.float32)
    return x2[:, :ENT_DIM] * (1.0 - par) + x2[:, ENT_DIM:] * par


def _tc_body(rpos_s, rneg_s, proj_ref, rel_ref,
             hp_ref, tp_ref, hn_ref, tn_ref,
             hpi_ref, tpi_ref, hni_ref, tni_ref,
             ph_ref, pt_ref, nh_ref, nt_ref, pos_ref, neg_ref,
             pp_scr, pn_scr, rp_scr, rn_scr):
    i = pl.program_id(0)

    def gather_body(n, carry):
        rp = rpos_s[i * NB + n]
        rn = rneg_s[i * NB + n]
        pp_scr[n] = proj_ref[rp]
        pn_scr[n] = proj_ref[rn]
        rp_scr[n] = rel_ref[rp]
        rn_scr[n] = rel_ref[rn]
        return carry

    lax.fori_loop(0, NB, gather_body, 0)

    hp = _sel_half(hp_ref[...], hpi_ref[...])
    tp = _sel_half(tp_ref[...], tpi_ref[...])
    hn = _sel_half(hn_ref[...], hni_ref[...])
    tn = _sel_half(tn_ref[...], tni_ref[...])

    def matvec(p, e):
        e2 = jnp.concatenate([e, e], axis=1)
        prod = p * e2[:, None, :]
        lo = jnp.sum(prod[:, :, :ENT_DIM], axis=2)
        hi = jnp.sum(prod[:, :, ENT_DIM:], axis=2)
        return jnp.concatenate([lo, hi], axis=1)

    ppv = pp_scr[...]
    pnv = pn_scr[...]
    ph = matvec(ppv, hp)
    pt = matvec(ppv, tp)
    nh = matvec(pnv, hn)
    nt = matvec(pnv, tn)
    ph_ref[...] = ph
    pt_ref[...] = pt
    nh_ref[...] = nh
    nt_ref[...] = nt
    pos_ref[...] = jnp.sum(jnp.abs(ph + rp_scr[...] - pt), axis=1)
    neg_ref[...] = jnp.sum(jnp.abs(nh + rn_scr[...] - nt), axis=1)


def _tc_compute(pos_r, neg_r, proj3, rel_w,
                hp2, tp2, hn2, tn2, hpi, tpi, hni, tni, interpret=False):
    b = pos_r.shape[0]
    grid = (b // NB,)
    pair_spec = pl.BlockSpec((NB, 2 * ENT_DIM), lambda i, *_: (i, 0))
    par_spec = pl.BlockSpec((NB, 1), lambda i, *_: (i, 0))
    vec_spec = pl.BlockSpec((NB, ENT_DIM), lambda i, *_: (i, 0))
    out_shapes = (
        jax.ShapeDtypeStruct((b, ENT_DIM), jnp.float32),
        jax.ShapeDtypeStruct((b, ENT_DIM), jnp.float32),
        jax.ShapeDtypeStruct((b, ENT_DIM), jnp.float32),
        jax.ShapeDtypeStruct((b, ENT_DIM), jnp.float32),
        jax.ShapeDtypeStruct((b,), jnp.float32),
        jax.ShapeDtypeStruct((b,), jnp.float32),
    )
    return pl.pallas_call(
        _tc_body,
        grid_spec=pltpu.PrefetchScalarGridSpec(
            num_scalar_prefetch=2,
            grid=grid,
            in_specs=[
                pl.BlockSpec((proj3.shape[0], ENT_DIM // 2, 2 * ENT_DIM),
                             lambda i, *_: (0, 0, 0)),
                pl.BlockSpec((rel_w.shape[0], ENT_DIM),
                             lambda i, *_: (0, 0)),
                pair_spec, pair_spec, pair_spec, pair_spec,
                par_spec, par_spec, par_spec, par_spec,
            ],
            out_specs=[
                vec_spec, vec_spec, vec_spec, vec_spec,
                pl.BlockSpec((NB,), lambda i, *_: (i,)),
                pl.BlockSpec((NB,), lambda i, *_: (i,)),
            ],
            scratch_shapes=[
                pltpu.VMEM((NB, ENT_DIM // 2, 2 * ENT_DIM), jnp.float32),
                pltpu.VMEM((NB, ENT_DIM // 2, 2 * ENT_DIM), jnp.float32),
                pltpu.VMEM((NB, ENT_DIM), jnp.float32),
                pltpu.VMEM((NB, ENT_DIM), jnp.float32),
            ],
        ),
        out_shape=out_shapes,
        interpret=interpret,
    )(pos_r, neg_r, proj3, rel_w, hp2, tp2, hn2, tn2, hpi, tpi, hni, tni)


def kernel(pos_h, pos_t, pos_r, neg_h, neg_t, neg_r, ent_w, rel_w, proj_w):
    b = pos_h.shape[0]
    ent_idx = jnp.concatenate(
        [pos_h, pos_t, neg_h, neg_t]).astype(jnp.int32)
    ent2 = ent_w.reshape(ent_w.shape[0] // 2, 2 * ENT_DIM)
    ent_g2 = _sc_gather(ent2, ent_idx // 2)
    hp2 = ent_g2[0 * b:1 * b]
    tp2 = ent_g2[1 * b:2 * b]
    hn2 = ent_g2[2 * b:3 * b]
    tn2 = ent_g2[3 * b:4 * b]
    proj3 = proj_w.reshape(proj_w.shape[0], 2, ENT_DIM // 2, ENT_DIM).transpose(
        0, 2, 1, 3).reshape(proj_w.shape[0], ENT_DIM // 2, 2 * ENT_DIM)
    ph, pt, nh, nt, pos, neg = _tc_compute(
        pos_r.astype(jnp.int32), neg_r.astype(jnp.int32),
        proj3, rel_w,
        hp2, tp2, hn2, tn2,
        pos_h.astype(jnp.int32).reshape(b, 1),
        pos_t.astype(jnp.int32).reshape(b, 1),
        neg_h.astype(jnp.int32).reshape(b, 1),
        neg_t.astype(jnp.int32).reshape(b, 1))
    return (pos, neg, ph, pt, nh, nt)
